# baseline (speedup 1.0000x reference)
import jax
import jax.numpy as jnp
from jax.experimental import pallas as pl
from jax.experimental.pallas import tpu as pltpu

_BLOCK_B = 32768


def _anet_fused_kernel(x_ref, w1t_ref, b1_ref, w2t_ref, b2_ref, o_ref):
    h = jax.lax.dot_general(
        x_ref[...].astype(jnp.bfloat16), w1t_ref[...],
        (((1,), (1,)), ((), ())), preferred_element_type=jnp.float32)
    h = jnp.maximum(h + b1_ref[...], 0.0)
    y = jax.lax.dot_general(
        h.astype(jnp.bfloat16), w2t_ref[...],
        (((1,), (1,)), ((), ())), preferred_element_type=jnp.float32)
    y = jnp.tanh(y + b2_ref[...]) * 2.0
    o_ref[...] = y.T


def kernel(x, w1, b1, w2, b2):
    B, s_dim = x.shape
    hidden = w1.shape[1]
    a_dim = w2.shape[1]
    x = x.astype(jnp.float32)
    w1t = jnp.transpose(w1).astype(jnp.bfloat16)
    w2t = jnp.transpose(w2).astype(jnp.bfloat16)
    b1 = jnp.reshape(b1, (1, hidden)).astype(jnp.float32)
    b2 = jnp.reshape(b2, (1, a_dim)).astype(jnp.float32)

    block_b = min(_BLOCK_B, B)
    pad_b = (-B) % (block_b if B > block_b else 8)
    if pad_b:
        x = jnp.pad(x, ((0, pad_b), (0, 0)))
    bp = B + pad_b
    block_b = min(block_b, bp)
    nb = bp // block_b

    out_t = pl.pallas_call(
        _anet_fused_kernel,
        out_shape=jax.ShapeDtypeStruct((a_dim, bp), jnp.float32),
        grid=(nb,),
        in_specs=[
            pl.BlockSpec((block_b, s_dim), lambda i: (i, 0)),
            pl.BlockSpec((hidden, s_dim), lambda i: (0, 0)),
            pl.BlockSpec((1, hidden), lambda i: (0, 0)),
            pl.BlockSpec((a_dim, hidden), lambda i: (0, 0)),
            pl.BlockSpec((1, a_dim), lambda i: (0, 0)),
        ],
        out_specs=pl.BlockSpec((a_dim, block_b), lambda i: (0, i)),
        compiler_params=pltpu.CompilerParams(
            dimension_semantics=("arbitrary",)),
    )(x, w1t, b1, w2t, b2)

    return out_t[:, :B].T


# speedup vs baseline: 1.0889x; 1.0889x over previous
import jax
import jax.numpy as jnp
from jax.experimental import pallas as pl
from jax.experimental.pallas import tpu as pltpu

_BLOCK_B = 32768


def _anet_dual_kernel(xa_ref, xb_ref, w1t_ref, b1_ref, w2t_ref, b2_ref,
                      o_ref):
    half = xa_ref.shape[0]
    for idx, xr in enumerate((xa_ref, xb_ref)):
        h = jax.lax.dot_general(
            xr[...], w1t_ref[...], (((1,), (1,)), ((), ())),
            preferred_element_type=jnp.float32)
        h = jnp.maximum(h + b1_ref[...], 0.0)
        y = jax.lax.dot_general(
            h, w2t_ref[...], (((1,), (1,)), ((), ())),
            preferred_element_type=jnp.float32)
        y = jnp.tanh(y + b2_ref[...]) * 2.0
        o_ref[:, idx * half:(idx + 1) * half] = y.T


def kernel(x, w1, b1, w2, b2):
    B, s_dim = x.shape
    hidden = w1.shape[1]
    a_dim = w2.shape[1]
    x = x.astype(jnp.float32)
    w1t = jnp.transpose(w1).astype(jnp.float32)
    w2t = jnp.transpose(w2).astype(jnp.float32)
    b1 = jnp.reshape(b1, (1, hidden)).astype(jnp.float32)
    b2 = jnp.reshape(b2, (1, a_dim)).astype(jnp.float32)

    block_b = min(_BLOCK_B, ((B + 15) // 16) * 16)
    pad_b = (-B) % block_b
    if pad_b:
        x = jnp.pad(x, ((0, pad_b), (0, 0)))
    bp = B + pad_b
    nb = bp // block_b
    half = block_b // 2

    out_t = pl.pallas_call(
        _anet_dual_kernel,
        out_shape=jax.ShapeDtypeStruct((a_dim, bp), jnp.float32),
        grid=(nb,),
        in_specs=[
            pl.BlockSpec((half, s_dim), lambda i: (2 * i, 0)),
            pl.BlockSpec((half, s_dim), lambda i: (2 * i + 1, 0)),
            pl.BlockSpec((hidden, s_dim), lambda i: (0, 0)),
            pl.BlockSpec((1, hidden), lambda i: (0, 0)),
            pl.BlockSpec((a_dim, hidden), lambda i: (0, 0)),
            pl.BlockSpec((1, a_dim), lambda i: (0, 0)),
        ],
        out_specs=pl.BlockSpec((a_dim, block_b), lambda i: (0, i)),
        compiler_params=pltpu.CompilerParams(
            dimension_semantics=("arbitrary",)),
    )(x, x, w1t, b1, w2t, b2)

    return out_t[:, :B].T


# speedup vs baseline: 1.0998x; 1.0100x over previous
import jax
import jax.numpy as jnp
from jax.experimental import pallas as pl
from jax.experimental.pallas import tpu as pltpu

_BLOCK_B = 32768


def _anet_fused_kernel(x_ref, w1t_ref, b1_ref, w2t_ref, b2_ref, o_ref):
    h = jax.lax.dot_general(
        x_ref[...], w1t_ref[...],
        (((1,), (1,)), ((), ())), preferred_element_type=jnp.float32)
    h = jnp.maximum(h + b1_ref[...], 0.0)
    y = jax.lax.dot_general(
        h, w2t_ref[...],
        (((1,), (1,)), ((), ())), preferred_element_type=jnp.float32)
    y = jnp.tanh(y + b2_ref[...]) * 2.0
    o_ref[...] = y.T


def kernel(x, w1, b1, w2, b2):
    B, s_dim = x.shape
    hidden = w1.shape[1]
    a_dim = w2.shape[1]
    x = x.astype(jnp.float32)
    w1t = jnp.transpose(w1).astype(jnp.float32)
    w2t = jnp.transpose(w2).astype(jnp.float32)
    b1 = jnp.reshape(b1, (1, hidden)).astype(jnp.float32)
    b2 = jnp.reshape(b2, (1, a_dim)).astype(jnp.float32)

    block_b = min(_BLOCK_B, B)
    pad_b = (-B) % (block_b if B > block_b else 8)
    if pad_b:
        x = jnp.pad(x, ((0, pad_b), (0, 0)))
    bp = B + pad_b
    block_b = min(block_b, bp)
    nb = bp // block_b

    out_t = pl.pallas_call(
        _anet_fused_kernel,
        out_shape=jax.ShapeDtypeStruct((a_dim, bp), jnp.float32),
        grid=(nb,),
        in_specs=[
            pl.BlockSpec((block_b, s_dim), lambda i: (i, 0)),
            pl.BlockSpec((hidden, s_dim), lambda i: (0, 0)),
            pl.BlockSpec((1, hidden), lambda i: (0, 0)),
            pl.BlockSpec((a_dim, hidden), lambda i: (0, 0)),
            pl.BlockSpec((1, a_dim), lambda i: (0, 0)),
        ],
        out_specs=pl.BlockSpec((a_dim, block_b), lambda i: (0, i)),
        compiler_params=pltpu.CompilerParams(
            dimension_semantics=("arbitrary",)),
    )(x, w1t, b1, w2t, b2)

    return out_t[:, :B].T
